# baseline (device time: 81801 ns/iter reference)
import jax
import jax.numpy as jnp
from jax import lax
from jax.experimental import pallas as pl
from jax.experimental.pallas import tpu as pltpu

try:
    jax.block_until_ready(jax.jit(lambda v: v + 1)(jnp.zeros((8, 128), jnp.float32)))
except Exception:
    pass

W = 32
B = 2
SQ = 512
SKV = 512
HL = 8
DH = 64
DM = 768
ROWS = B * SQ
CH = ROWS // W
HD = HL * DH


def kernel(x, Wq, K_ext, V_ext, Wo):
    me = lax.axis_index("i")
    wq_i = lax.dynamic_slice(Wq, (0, me * HD), (DM, HD)).astype(jnp.bfloat16)
    wo_i = lax.dynamic_slice(Wo, (me * HD, 0), (HD, DM)).astype(jnp.bfloat16)
    x2 = x.reshape(ROWS, DM).astype(jnp.bfloat16)
    k_b = K_ext.astype(jnp.bfloat16)
    v_b = V_ext.astype(jnp.bfloat16)

    def body(x_ref, wq_ref, k_ref, v_ref, wo_ref, out_ref,
             partial_ref, rs_buf, rs_send, rs_recv, ag_send, ag_recv):
        me = lax.axis_index("i")

        bar = pltpu.get_barrier_semaphore()
        for k in range(1, W):
            pl.semaphore_signal(bar, inc=1, device_id=((me + k) % W,),
                                device_id_type=pl.DeviceIdType.MESH)
        pl.semaphore_wait(bar, W - 1)

        rs_rdmas = []
        for k in range(1, W):
            dst = (me + k) % W
            rs_rdmas.append(pltpu.make_async_remote_copy(
                src_ref=partial_ref.at[pl.ds(dst * CH, CH)],
                dst_ref=rs_buf.at[k],
                send_sem=rs_send.at[k],
                recv_sem=rs_recv.at[k],
                device_id=(dst,),
                device_id_type=pl.DeviceIdType.MESH,
            ))

        q_all = jnp.dot(x_ref[:, :], wq_ref[:, :],
                        preferred_element_type=jnp.float32)
        q_all = q_all.astype(jnp.bfloat16)
        qi = lax.broadcasted_iota(jnp.int32, (SQ, SKV), 0)
        ki = lax.broadcasted_iota(jnp.int32, (SQ, SKV), 1)
        mask = (jnp.abs(qi - ki) <= 128) | (ki < 32) | (qi < 32)
        chunks_per_b = SQ // CH
        for b in range(B):
            acc = jnp.zeros((SQ, DM), jnp.float32)
            for h in range(HL):
                q = q_all[b * SQ:(b + 1) * SQ, h * DH:(h + 1) * DH]
                kk = k_ref[b, :, h, :]
                s = lax.dot_general(q, kk, (((1,), (1,)), ((), ())),
                                    preferred_element_type=jnp.float32)
                s = s * 0.125
                w = jnp.exp(jnp.where(mask, s, -1e9))
                w = w / jnp.sum(w, axis=1, keepdims=True)
                ctx = jnp.dot(w.astype(jnp.bfloat16), v_ref[b, :, h, :],
                              preferred_element_type=jnp.float32)
                acc = acc + jnp.dot(ctx.astype(jnp.bfloat16),
                                    wo_ref[h * DH:(h + 1) * DH, :],
                                    preferred_element_type=jnp.float32)
            partial_ref[b * SQ:(b + 1) * SQ, :] = acc.astype(jnp.bfloat16)
            for k in range(1, W):
                dst = (me + k) % W
                in_this_b = (dst // chunks_per_b) == b

                @pl.when(in_this_b)
                def _(rdma=rs_rdmas[k - 1]):
                    rdma.start()

        red = partial_ref[pl.ds(me * CH, CH), :].astype(jnp.float32)
        for k in range(1, W):
            rs_rdmas[k - 1].wait_recv()
            red = red + rs_buf[k].astype(jnp.float32)

        out_ref[pl.ds(me * CH, CH), :] = red.astype(jnp.bfloat16)
        ag_rdmas = []
        for k in range(1, W):
            dst = (me + k) % W
            rdma = pltpu.make_async_remote_copy(
                src_ref=out_ref.at[pl.ds(me * CH, CH)],
                dst_ref=out_ref.at[pl.ds(me * CH, CH)],
                send_sem=ag_send.at[k],
                recv_sem=ag_recv.at[k],
                device_id=(dst,),
                device_id_type=pl.DeviceIdType.MESH,
            )
            rdma.start()
            ag_rdmas.append(rdma)

        for k in range(1, W):
            src = (me + W - k) % W
            recv = pltpu.make_async_remote_copy(
                src_ref=out_ref.at[pl.ds(me * CH, CH)],
                dst_ref=out_ref.at[pl.ds(src * CH, CH)],
                send_sem=ag_send.at[k],
                recv_sem=ag_recv.at[k],
                device_id=(src,),
                device_id_type=pl.DeviceIdType.MESH,
            )
            recv.wait_recv()

        for rdma in rs_rdmas:
            rdma.wait_send()
        for rdma in ag_rdmas:
            rdma.wait_send()

    out2 = pl.pallas_call(
        body,
        out_shape=jax.ShapeDtypeStruct((ROWS, DM), jnp.bfloat16),
        in_specs=[pl.BlockSpec(memory_space=pltpu.VMEM)] * 5,
        out_specs=pl.BlockSpec(memory_space=pltpu.VMEM),
        scratch_shapes=[
            pltpu.VMEM((ROWS, DM), jnp.bfloat16),
            pltpu.VMEM((W, CH, DM), jnp.bfloat16),
            pltpu.SemaphoreType.DMA((W,)),
            pltpu.SemaphoreType.DMA((W,)),
            pltpu.SemaphoreType.DMA((W,)),
            pltpu.SemaphoreType.DMA((W,)),
        ],
        compiler_params=pltpu.CompilerParams(collective_id=0),
    )(x2, wq_i, k_b, v_b, wo_i)
    return out2.reshape(B, SQ, DM).astype(jnp.float32)


# device time: 73049 ns/iter; 1.1198x vs baseline; 1.1198x over previous
import jax
import jax.numpy as jnp
from jax import lax
from jax.experimental import pallas as pl
from jax.experimental.pallas import tpu as pltpu

try:
    jax.block_until_ready(jax.jit(lambda v: v + 1)(jnp.zeros((8, 128), jnp.float32)))
except Exception:
    pass

W = 32
B = 2
SQ = 512
SKV = 512
HL = 8
DH = 64
DM = 768
ROWS = B * SQ
CH = ROWS // W
HD = HL * DH


def kernel(x, Wq, K_ext, V_ext, Wo):
    me = lax.axis_index("i")
    wq_i = lax.dynamic_slice(Wq, (0, me * HD), (DM, HD)).astype(jnp.bfloat16)
    wo_i = lax.dynamic_slice(Wo, (me * HD, 0), (HD, DM)).astype(jnp.bfloat16)
    x2 = x.reshape(ROWS, DM).astype(jnp.bfloat16)
    k_b = K_ext.astype(jnp.bfloat16)
    v_b = V_ext.astype(jnp.bfloat16)

    def body(x_ref, wq_ref, k_ref, v_ref, wo_ref, out_ref,
             partial_ref, rs_buf, rs_send, rs_recv, ag_send, ag_recv):
        me = lax.axis_index("i")

        bar = pltpu.get_barrier_semaphore()
        for k in range(1, W):
            pl.semaphore_signal(bar, inc=1, device_id=((me + k) % W,),
                                device_id_type=pl.DeviceIdType.MESH)
        pl.semaphore_wait(bar, W - 1)

        rs_rdmas = []
        for k in range(1, W):
            dst = (me + k) % W
            rs_rdmas.append(pltpu.make_async_remote_copy(
                src_ref=partial_ref.at[pl.ds(dst * CH, CH)],
                dst_ref=rs_buf.at[k],
                send_sem=rs_send.at[k],
                recv_sem=rs_recv.at[k],
                device_id=(dst,),
                device_id_type=pl.DeviceIdType.MESH,
            ))

        q_all = jnp.dot(x_ref[:, :], wq_ref[:, :],
                        preferred_element_type=jnp.float32)
        q_all = q_all.astype(jnp.bfloat16)
        qi = lax.broadcasted_iota(jnp.int32, (SQ, SKV), 0)
        ki = lax.broadcasted_iota(jnp.int32, (SQ, SKV), 1)
        mask = (jnp.abs(qi - ki) <= 128) | (ki < 32) | (qi < 32)
        for b in range(B):
            acc = jnp.zeros((SQ, DM), jnp.float32)
            for h in range(HL):
                q = q_all[b * SQ:(b + 1) * SQ, h * DH:(h + 1) * DH]
                kk = k_ref[b, :, h, :]
                s = lax.dot_general(q, kk, (((1,), (1,)), ((), ())),
                                    preferred_element_type=jnp.float32)
                s = s * 0.125
                w = jnp.exp(jnp.where(mask, s, -1e9)).astype(jnp.bfloat16)
                denom = jnp.sum(w, axis=1, keepdims=True,
                                dtype=jnp.float32)
                ctx = jnp.dot(w, v_ref[b, :, h, :],
                              preferred_element_type=jnp.float32)
                ctx = ctx / denom
                acc = acc + jnp.dot(ctx.astype(jnp.bfloat16),
                                    wo_ref[h * DH:(h + 1) * DH, :],
                                    preferred_element_type=jnp.float32)
            partial_ref[b * SQ:(b + 1) * SQ, :] = acc.astype(jnp.bfloat16)

        for rdma in rs_rdmas:
            rdma.start()

        red = partial_ref[pl.ds(me * CH, CH), :].astype(jnp.float32)
        for k in range(1, W):
            rs_rdmas[k - 1].wait_recv()
            red = red + rs_buf[k].astype(jnp.float32)

        out_ref[pl.ds(me * CH, CH), :] = red.astype(jnp.bfloat16)
        ag_rdmas = []
        for k in range(1, W):
            dst = (me + k) % W
            rdma = pltpu.make_async_remote_copy(
                src_ref=out_ref.at[pl.ds(me * CH, CH)],
                dst_ref=out_ref.at[pl.ds(me * CH, CH)],
                send_sem=ag_send.at[k],
                recv_sem=ag_recv.at[k],
                device_id=(dst,),
                device_id_type=pl.DeviceIdType.MESH,
            )
            rdma.start()
            ag_rdmas.append(rdma)

        for k in range(1, W):
            src = (me + W - k) % W
            recv = pltpu.make_async_remote_copy(
                src_ref=out_ref.at[pl.ds(me * CH, CH)],
                dst_ref=out_ref.at[pl.ds(src * CH, CH)],
                send_sem=ag_send.at[k],
                recv_sem=ag_recv.at[k],
                device_id=(src,),
                device_id_type=pl.DeviceIdType.MESH,
            )
            recv.wait_recv()

        for rdma in rs_rdmas:
            rdma.wait_send()
        for rdma in ag_rdmas:
            rdma.wait_send()

    out2 = pl.pallas_call(
        body,
        out_shape=jax.ShapeDtypeStruct((ROWS, DM), jnp.bfloat16),
        in_specs=[pl.BlockSpec(memory_space=pltpu.VMEM)] * 5,
        out_specs=pl.BlockSpec(memory_space=pltpu.VMEM),
        scratch_shapes=[
            pltpu.VMEM((ROWS, DM), jnp.bfloat16),
            pltpu.VMEM((W, CH, DM), jnp.bfloat16),
            pltpu.SemaphoreType.DMA((W,)),
            pltpu.SemaphoreType.DMA((W,)),
            pltpu.SemaphoreType.DMA((W,)),
            pltpu.SemaphoreType.DMA((W,)),
        ],
        compiler_params=pltpu.CompilerParams(collective_id=0),
    )(x2, wq_i, k_b, v_b, wo_i)
    return out2.reshape(B, SQ, DM).astype(jnp.float32)
